# Initial kernel scaffold; baseline (speedup 1.0000x reference)
import jax
import jax.numpy as jnp
from jax.experimental import pallas as pl


def kernel(states, adjacency, target_idx, intervention_value, W1, b1, W2, b2):
    raise NotImplementedError("write your pallas kernel here")


# speedup vs baseline: 1.3457x; 1.3457x over previous
import functools

import jax
import jax.numpy as jnp
from jax.experimental import pallas as pl
from jax.experimental.pallas import tpu as pltpu


def _fused_kernel(tgt_ref, st_ref, adj_ref, iv_ref,
                  w1src_ref, w1st_ref, w1w_ref, b1_ref, w2_ref, b2_ref,
                  cf_pred_ref, fact_ref, cf_adj_ref):
    bb, n, d = st_ref.shape
    tgt = tgt_ref[0, 0, :]
    st = st_ref[...]
    adj = adj_ref[...]
    iv = iv_ref[...]

    n_iota = jax.lax.broadcasted_iota(jnp.int32, (bb, n), 1)
    onehot = n_iota == tgt[:, None]
    onehot_f = onehot.astype(jnp.float32)

    cf_adj_ref[...] = adj * (1.0 - onehot_f)[:, None, :]
    edge_w = jnp.sum(adj * onehot_f[:, :, None], axis=1)

    fact_ref[...] = st

    iv_proj = jnp.dot(iv, w1src_ref[...],
                      preferred_element_type=jnp.float32)
    st_proj = jnp.dot(st.reshape(bb * n, d), w1st_ref[...],
                      preferred_element_type=jnp.float32).reshape(bb, n, d)
    pre = (st_proj + iv_proj[:, None, :]
           + edge_w[:, :, None] * w1w_ref[0, :] + b1_ref[0, :])
    h = jnp.maximum(pre, 0.0)
    h2 = jnp.dot(h.reshape(bb * n, d), w2_ref[...],
                 preferred_element_type=jnp.float32).reshape(bb, n, d)
    slot = jnp.tanh(h2 + b2_ref[0, :])
    effects = slot * edge_w[:, :, None]
    oh3 = onehot_f[:, :, None]
    cf_states = st + oh3 * (iv[:, None, :] - st)
    cf_pred_ref[...] = cf_states + effects


@functools.partial(jax.jit, static_argnames=("interpret",))
def kernel(states, adjacency, target_idx, intervention_value,
           W1, b1, W2, b2, interpret=False):
    B, N, D = states.shape
    BB = 128
    nb = B // BB

    tgt3 = target_idx.astype(jnp.int32).reshape(nb, 1, BB)
    w1src = W1[:D]
    w1st = W1[D:2 * D]
    w1w = W1[2 * D].reshape(1, D)
    b1r = b1.reshape(1, D)
    b2r = b2.reshape(1, D)

    grid = (nb,)
    cf_pred, fact, cf_adj = pl.pallas_call(
        _fused_kernel,
        grid=grid,
        in_specs=[
            pl.BlockSpec((1, 1, BB), lambda i: (i, 0, 0)),
            pl.BlockSpec((BB, N, D), lambda i: (i, 0, 0)),
            pl.BlockSpec((BB, N, N), lambda i: (i, 0, 0)),
            pl.BlockSpec((BB, D), lambda i: (i, 0)),
            pl.BlockSpec((D, D), lambda i: (0, 0)),
            pl.BlockSpec((D, D), lambda i: (0, 0)),
            pl.BlockSpec((1, D), lambda i: (0, 0)),
            pl.BlockSpec((1, D), lambda i: (0, 0)),
            pl.BlockSpec((D, D), lambda i: (0, 0)),
            pl.BlockSpec((1, D), lambda i: (0, 0)),
        ],
        out_specs=[
            pl.BlockSpec((BB, N, D), lambda i: (i, 0, 0)),
            pl.BlockSpec((BB, N, D), lambda i: (i, 0, 0)),
            pl.BlockSpec((BB, N, N), lambda i: (i, 0, 0)),
        ],
        out_shape=[
            jax.ShapeDtypeStruct((B, N, D), jnp.float32),
            jax.ShapeDtypeStruct((B, N, D), jnp.float32),
            jax.ShapeDtypeStruct((B, N, N), jnp.float32),
        ],
        compiler_params=pltpu.CompilerParams(
            dimension_semantics=("arbitrary",),
        ),
        interpret=interpret,
    )(tgt3, states, adjacency, intervention_value,
      w1src, w1st, w1w, b1r, W2, b2r)

    return (cf_pred.reshape(B, 1, N, D), fact.reshape(B, 1, N, D),
            cf_adj, target_idx, intervention_value)


# speedup vs baseline: 9.3293x; 6.9325x over previous
import functools

import jax
import jax.numpy as jnp
from jax.experimental import pallas as pl
from jax.experimental.pallas import tpu as pltpu


def _adj_kernel(tgt_ref, adj_ref, cf_adj_ref, edge_ref):
    bi, n, b = adj_ref.shape
    t = tgt_ref[...]
    tb = jnp.broadcast_to(t, (n, b))
    j_iota = jax.lax.broadcasted_iota(jnp.int32, (n, b), 0)
    keep = (j_iota != tb).astype(jnp.float32)

    adj = adj_ref[...]
    cf_adj_ref[...] = adj * keep[None, :, :]

    @pl.when(pl.program_id(0) == 0)
    def _():
        edge_ref[...] = jnp.zeros_like(edge_ref)

    i0 = pl.program_id(0) * bi
    acc = edge_ref[...]
    for li in range(bi):
        rowmask = (t == (i0 + li)).astype(jnp.float32)
        acc = acc + adj[li] * rowmask
    edge_ref[...] = acc


def _mlp_kernel(tgt_ref, st_ref, edge_ref, iv_ref,
                w1srcT_ref, w1stT_ref, w1w_ref, b1_ref, w2T_ref, b2_ref,
                cf_pred_ref, fact_ref):
    bn, d, b = st_ref.shape
    t = tgt_ref[...]
    iv = iv_ref[...]
    ivp = (jnp.dot(w1srcT_ref[...], iv,
                   preferred_element_type=jnp.float32) + b1_ref[...])
    n0 = pl.program_id(0) * bn
    for ln in range(bn):
        stn = st_ref[ln]
        ew = edge_ref[ln:ln + 1, :]
        pre = (jnp.dot(w1stT_ref[...], stn,
                       preferred_element_type=jnp.float32)
               + ivp + w1w_ref[...] * ew)
        h = jnp.maximum(pre, 0.0)
        slot = jnp.tanh(jnp.dot(w2T_ref[...], h,
                                preferred_element_type=jnp.float32)
                        + b2_ref[...])
        m = (t == (n0 + ln)).astype(jnp.float32)
        cf_pred_ref[ln] = stn + m * (iv - stn) + slot * ew
        fact_ref[ln] = stn


@functools.partial(jax.jit, static_argnames=("interpret",))
def kernel(states, adjacency, target_idx, intervention_value,
           W1, b1, W2, b2, interpret=False):
    B, N, D = states.shape
    BI = 8
    BN = 8

    st_t = jnp.transpose(states, (1, 2, 0))
    adj_t = jnp.transpose(adjacency, (1, 2, 0))
    iv_t = intervention_value.T
    tgt2 = target_idx.astype(jnp.int32).reshape(1, B)
    W1T = W1.T
    w1srcT = W1T[:, :D]
    w1stT = W1T[:, D:2 * D]
    w1w = W1T[:, 2 * D:2 * D + 1]
    b1c = b1.reshape(D, 1)
    b2c = b2.reshape(D, 1)
    W2T = W2.T

    cf_adj_t, edge_w = pl.pallas_call(
        _adj_kernel,
        grid=(N // BI,),
        in_specs=[
            pl.BlockSpec((1, B), lambda i: (0, 0)),
            pl.BlockSpec((BI, N, B), lambda i: (i, 0, 0)),
        ],
        out_specs=[
            pl.BlockSpec((BI, N, B), lambda i: (i, 0, 0)),
            pl.BlockSpec((N, B), lambda i: (0, 0)),
        ],
        out_shape=[
            jax.ShapeDtypeStruct((N, N, B), jnp.float32),
            jax.ShapeDtypeStruct((N, B), jnp.float32),
        ],
        compiler_params=pltpu.CompilerParams(
            dimension_semantics=("arbitrary",),
        ),
        interpret=interpret,
    )(tgt2, adj_t)

    cf_pred_t, fact_t = pl.pallas_call(
        _mlp_kernel,
        grid=(N // BN,),
        in_specs=[
            pl.BlockSpec((1, B), lambda i: (0, 0)),
            pl.BlockSpec((BN, D, B), lambda i: (i, 0, 0)),
            pl.BlockSpec((BN, B), lambda i: (i, 0)),
            pl.BlockSpec((D, B), lambda i: (0, 0)),
            pl.BlockSpec((D, D), lambda i: (0, 0)),
            pl.BlockSpec((D, D), lambda i: (0, 0)),
            pl.BlockSpec((D, 1), lambda i: (0, 0)),
            pl.BlockSpec((D, 1), lambda i: (0, 0)),
            pl.BlockSpec((D, D), lambda i: (0, 0)),
            pl.BlockSpec((D, 1), lambda i: (0, 0)),
        ],
        out_specs=[
            pl.BlockSpec((BN, D, B), lambda i: (i, 0, 0)),
            pl.BlockSpec((BN, D, B), lambda i: (i, 0, 0)),
        ],
        out_shape=[
            jax.ShapeDtypeStruct((N, D, B), jnp.float32),
            jax.ShapeDtypeStruct((N, D, B), jnp.float32),
        ],
        compiler_params=pltpu.CompilerParams(
            dimension_semantics=("arbitrary",),
        ),
        interpret=interpret,
    )(tgt2, st_t, edge_w, iv_t, w1srcT, w1stT, w1w, b1c, W2T, b2c)

    cf_pred = jnp.transpose(cf_pred_t, (2, 0, 1)).reshape(B, 1, N, D)
    fact = jnp.transpose(fact_t, (2, 0, 1)).reshape(B, 1, N, D)
    cf_adj = jnp.transpose(cf_adj_t, (2, 0, 1))
    return (cf_pred, fact, cf_adj, target_idx, intervention_value)
